# baseline (speedup 1.0000x reference)
import jax
import jax.numpy as jnp
from jax.experimental import pallas as pl

_SA = [
    (1024, 0.05, 16),
    (512, 0.1, 16),
    (256, 0.2, 16),
    (128, 0.4, 16),
]


_INTERPRET = False


def _fps_body(x_ref, idx_ref, nx_ref, ny_ref, nz_ref, *, npoint, n):
    xs = x_ref[:, 0, :]
    ys = x_ref[:, 1, :]
    zs = x_ref[:, 2, :]
    b = xs.shape[0]
    iota = jax.lax.broadcasted_iota(jnp.int32, (b, n), 1)
    lane128 = jax.lax.broadcasted_iota(jnp.int32, (b, 128), 1)

    def body(j, carry):
        dist, far, pi, px, py, pz = carry
        sel = iota == far
        cx = jnp.sum(jnp.where(sel, xs, 0.0), axis=1, keepdims=True)
        cy = jnp.sum(jnp.where(sel, ys, 0.0), axis=1, keepdims=True)
        cz = jnp.sum(jnp.where(sel, zs, 0.0), axis=1, keepdims=True)
        lane = lane128 == j
        pi = jnp.where(lane, far, pi)
        px = jnp.where(lane, cx, px)
        py = jnp.where(lane, cy, py)
        pz = jnp.where(lane, cz, pz)
        d = (xs - cx) ** 2 + (ys - cy) ** 2 + (zs - cz) ** 2
        dist = jnp.minimum(dist, d)
        m = jnp.max(dist, axis=1, keepdims=True)
        far2 = jnp.min(jnp.where(dist == m, iota, n), axis=1, keepdims=True)
        return dist, far2, pi, px, py, pz

    dist = jnp.full((b, n), 1e10, jnp.float32)
    far = jnp.zeros((b, 1), jnp.int32)
    p0 = idx_ref[:, 0:128]
    q0 = nx_ref[:, 0:128]
    for blk in range(npoint // 128):
        dist, far, pi, px, py, pz = jax.lax.fori_loop(
            0, 128, body, (dist, far, p0, q0, q0, q0))
        sl = pl.ds(blk * 128, 128)
        idx_ref[:, sl] = pi
        nx_ref[:, sl] = px
        ny_ref[:, sl] = py
        nz_ref[:, sl] = pz


def _fps_pallas(xyz, npoint):
    B, _, N = xyz.shape
    import functools
    fn = functools.partial(_fps_body, npoint=npoint, n=N)
    idx, nx, ny, nz = pl.pallas_call(
        fn,
        out_shape=[
            jax.ShapeDtypeStruct((B, npoint), jnp.int32),
            jax.ShapeDtypeStruct((B, npoint), jnp.float32),
            jax.ShapeDtypeStruct((B, npoint), jnp.float32),
            jax.ShapeDtypeStruct((B, npoint), jnp.float32),
        ],
        interpret=_INTERPRET,
    )(xyz)
    new_xyz = jnp.stack([nx, ny, nz], axis=-1)
    return idx, new_xyz


def _sqdist(src, dst):
    d = -2.0 * jnp.matmul(src, dst.transpose(0, 2, 1))
    d = d + jnp.sum(src ** 2, -1)[:, :, None]
    d = d + jnp.sum(dst ** 2, -1)[:, None, :]
    return d


def _index_points(points, idx):
    return jax.vmap(lambda p, i: p[i])(points, idx)


def _fps(xyz, npoint):
    B, N, _ = xyz.shape

    def body(i, state):
        centroids, distance, farthest = state
        centroids = centroids.at[:, i].set(farthest)
        centroid = jnp.take_along_axis(xyz, farthest[:, None, None], axis=1)
        dist = jnp.sum((xyz - centroid) ** 2, -1)
        distance = jnp.minimum(distance, dist)
        farthest = jnp.argmax(distance, axis=-1).astype(jnp.int32)
        return centroids, distance, farthest

    centroids = jnp.zeros((B, npoint), dtype=jnp.int32)
    distance = jnp.full((B, N), 1e10, dtype=jnp.float32)
    farthest = jnp.zeros((B,), dtype=jnp.int32)
    centroids, _, _ = jax.lax.fori_loop(0, npoint, body, (centroids, distance, farthest))
    return centroids


def _ball(radius, nsample, xyz, new_xyz):
    B, N, _ = xyz.shape
    S = new_xyz.shape[1]
    sqrdists = _sqdist(new_xyz, xyz)
    group_idx = jnp.broadcast_to(jnp.arange(N, dtype=jnp.int32), (B, S, N))
    group_idx = jnp.where(sqrdists > radius ** 2, N, group_idx)
    group_idx = jnp.sort(group_idx, axis=-1)[:, :, :nsample]
    group_first = jnp.broadcast_to(group_idx[:, :, :1], group_idx.shape)
    group_idx = jnp.where(group_idx == N, group_first, group_idx)
    return group_idx


def _bn(x):
    axes = (0,) + tuple(range(2, x.ndim))
    m = jnp.mean(x, axis=axes, keepdims=True)
    v = jnp.var(x, axis=axes, keepdims=True)
    return (x - m) * jax.lax.rsqrt(v + 1e-5)


def _sa(xyz, points, params, npoint, radius, nsample):
    xyz_t = xyz.transpose(0, 2, 1)
    points_t = points.transpose(0, 2, 1)
    fps_idx, new_xyz = _fps_pallas(xyz, npoint)
    idx = _ball(radius, nsample, xyz_t, new_xyz)
    grouped_xyz = _index_points(xyz_t, idx)
    grouped_xyz_norm = grouped_xyz - new_xyz[:, :, None, :]
    grouped_points = _index_points(points_t, idx)
    new_points = jnp.concatenate([grouped_xyz_norm, grouped_points], axis=-1)
    x = new_points.transpose(0, 3, 2, 1)
    for w, b in params:
        x = jnp.einsum('bcks,oc->boks', x, w) + b[None, :, None, None]
        x = jax.nn.relu(_bn(x))
    new_points_out = jnp.max(x, axis=2)
    return new_xyz.transpose(0, 2, 1), new_points_out


def _fp(xyz1, xyz2, points1, points2, params):
    xyz1_t = xyz1.transpose(0, 2, 1)
    xyz2_t = xyz2.transpose(0, 2, 1)
    points2_t = points2.transpose(0, 2, 1)
    dists = _sqdist(xyz1_t, xyz2_t)
    neg_d, idx = jax.lax.top_k(-dists, 3)
    d3 = -neg_d
    dist_recip = 1.0 / (d3 + 1e-8)
    norm = jnp.sum(dist_recip, axis=2, keepdims=True)
    weight = dist_recip / norm
    interpolated = jnp.sum(_index_points(points2_t, idx) * weight[..., None], axis=2)
    new_points = jnp.concatenate([points1.transpose(0, 2, 1), interpolated], axis=-1)
    x = new_points.transpose(0, 2, 1)
    for w, b in params:
        x = jnp.einsum('bcn,oc->bon', x, w) + b[None, :, None]
        x = jax.nn.relu(_bn(x))
    return x


def _ident_kernel(x_ref, o_ref):
    o_ref[...] = x_ref[...]


def kernel(xyz, sa1_params, sa2_params, sa3_params, sa4_params, fp4_params, fp3_params, fp2_params, fp1_params):
    l0_xyz = xyz
    l0_points = xyz
    l1_xyz, l1_points = _sa(l0_xyz, l0_points, sa1_params, *_SA[0])
    l2_xyz, l2_points = _sa(l1_xyz, l1_points, sa2_params, *_SA[1])
    l3_xyz, l3_points = _sa(l2_xyz, l2_points, sa3_params, *_SA[2])
    l4_xyz, l4_points = _sa(l3_xyz, l3_points, sa4_params, *_SA[3])
    l3_points = _fp(l3_xyz, l4_xyz, l3_points, l4_points, fp4_params)
    l2_points = _fp(l2_xyz, l3_xyz, l2_points, l3_points, fp3_params)
    l1_points = _fp(l1_xyz, l2_xyz, l1_points, l2_points, fp2_params)
    l0_points = _fp(l0_xyz, l1_xyz, l0_points, l1_points, fp1_params)
    out = pl.pallas_call(
        _ident_kernel,
        out_shape=jax.ShapeDtypeStruct(l0_points.shape, l0_points.dtype),
    )(l0_points)
    return out
